# Initial kernel scaffold; baseline (speedup 1.0000x reference)
import jax
import jax.numpy as jnp
from jax.experimental import pallas as pl


def kernel(world_pos, prev_world_pos, mesh_pos, node_type, edge_index, params):
    raise NotImplementedError("write your pallas kernel here")


# speedup vs baseline: 2.3036x; 2.3036x over previous
import functools

import jax
import jax.numpy as jnp
from jax import lax
from jax.experimental import pallas as pl
from jax.experimental.pallas import tpu as pltpu
from jax.experimental.pallas import tpu_sc as plsc

N_NODES = 10000
N_EDGES = 160000
N_TYPES = 9
LATENT = 128
N_BLOCKS = 15

NPAD = 10240
EPAD = 163840
CH = 128
CPW = 40
EPW = CH * CPW
NW = 32
TRASH = N_NODES
ROWS_PER_TILE = NPAD // 16

BLKE = 2048
BLKN = 2048

_F32 = jnp.float32


def _dot(a, b):
    return jnp.dot(a, b, preferred_element_type=jnp.float32)


def _mlp3_ln(h1pre, w2, b2, w3, b3):
    h = jnp.maximum(h1pre, 0.0)
    h = jnp.maximum(_dot(h, w2) + b2, 0.0)
    h = _dot(h, w3) + b3
    mu = jnp.mean(h, axis=-1, keepdims=True)
    var = jnp.mean((h - mu) * (h - mu), axis=-1, keepdims=True)
    return (h - mu) * lax.rsqrt(var + 1e-5)


def _enc_node_body(wpp, tpe, bwp, w1t, c1, w2, b2, w3, b3, w1s, w1r,
                   nl_o, ps_o, pr_o):
    cols = lax.broadcasted_iota(jnp.int32, (BLKN, 16), 1)
    oh = (cols == tpe[...]).astype(_F32)
    h1 = _dot(wpp[...], bwp[...]) + _dot(oh, w1t[...]) + c1[...]
    nl = _mlp3_ln(h1, w2[...], b2[...], w3[...], b3[...])
    nl_o[...] = nl
    ps_o[...] = _dot(nl, w1s[...])
    pr_o[...] = _dot(nl, w1r[...])


def _enc_edge_body(gs, gr, amat, wn, mn, ce, w2, b2, w3, b3, el_o):
    d = gs[...] - gr[...]
    lin = _dot(d, amat[...])
    d2 = d * d
    cols = lax.broadcasted_iota(jnp.int32, (1, 128), 1)
    w2sum = jnp.sum(jnp.where(cols < 3, d2, 0.0), axis=1, keepdims=True)
    m2sum = jnp.sum(jnp.where((cols >= 3) & (cols < 6), d2, 0.0),
                    axis=1, keepdims=True)
    h1 = lin + jnp.sqrt(w2sum) * wn[...] + jnp.sqrt(m2sum) * mn[...] + ce[...]
    el_o[...] = _mlp3_ln(h1, w2[...], b2[...], w3[...], b3[...])


def _edge_blk_body(el, gs, gr, w1, b1, w2, b2, w3, b3, ne_o, elo_o):
    e = el[...]
    h1 = _dot(e, w1[...]) + gs[...] + gr[...] + b1[...]
    ne = _mlp3_ln(h1, w2[...], b2[...], w3[...], b3[...])
    ne_o[...] = ne
    elo_o[...] = e + ne


def _node_blk_body_proj(nl, p0, p1, w1n, w1a, b1, w2, b2, w3, b3, w1s, w1r,
                        nlo_o, ps_o, pr_o):
    n = nl[...]
    agg = p0[...] + p1[...]
    h1 = _dot(n, w1n[...]) + _dot(agg, w1a[...]) + b1[...]
    nn = _mlp3_ln(h1, w2[...], b2[...], w3[...], b3[...])
    out = n + nn
    nlo_o[...] = out
    ps_o[...] = _dot(out, w1s[...])
    pr_o[...] = _dot(out, w1r[...])


def _node_blk_body_last(nl, p0, p1, w1n, w1a, b1, w2, b2, w3, b3, nlo_o):
    n = nl[...]
    agg = p0[...] + p1[...]
    h1 = _dot(n, w1n[...]) + _dot(agg, w1a[...]) + b1[...]
    nn = _mlp3_ln(h1, w2[...], b2[...], w3[...], b3[...])
    nlo_o[...] = n + nn


def _dec_body(nl, c1m, a1m, maskf, w1, b1, w2, b2, w3p, b3p, out_o):
    h = jnp.maximum(_dot(nl[...], w1[...]) + b1[...], 0.0)
    h = jnp.maximum(_dot(h, w2[...]) + b2[...], 0.0)
    acc8 = _dot(h, w3p[...]) + b3p[...]
    m = maskf[...]
    out_o[...] = m * (c1m[...] + acc8) + (1.0 - m) * a1m[...]


def _row_spec(blk, d):
    return pl.BlockSpec((blk, d), lambda i: (i, 0))


def _w_spec(r, c):
    return pl.BlockSpec((r, c), lambda i: (0, 0))


def _tc_call(body, grid, in_specs, out_specs, out_shape):
    return pl.pallas_call(
        body, grid=(grid,), in_specs=in_specs, out_specs=out_specs,
        out_shape=out_shape,
        compiler_params=pltpu.CompilerParams(
            dimension_semantics=("arbitrary",)),
    )


def _enc_node_call(wpp, tpe, bwp, w1t, c1, w2, b2, w3, b3, w1s, w1r):
    g = NPAD // BLKN
    specs = [_row_spec(BLKN, 8), _row_spec(BLKN, 1),
             _w_spec(8, 128), _w_spec(16, 128), _w_spec(1, 128),
             _w_spec(128, 128), _w_spec(1, 128), _w_spec(128, 128),
             _w_spec(1, 128), _w_spec(128, 128), _w_spec(128, 128)]
    outs = [_row_spec(BLKN, 128)] * 3
    shp = [jax.ShapeDtypeStruct((NPAD, 128), _F32)] * 3
    return _tc_call(_enc_node_body, g, specs, outs, shp)(
        wpp, tpe, bwp, w1t, c1, w2, b2, w3, b3, w1s, w1r)


def _enc_edge_call(gs, gr, amat, wn, mn, ce, w2, b2, w3, b3):
    g = EPAD // BLKE
    specs = [_row_spec(BLKE, 128), _row_spec(BLKE, 128),
             _w_spec(128, 128), _w_spec(1, 128), _w_spec(1, 128),
             _w_spec(1, 128), _w_spec(128, 128), _w_spec(1, 128),
             _w_spec(128, 128), _w_spec(1, 128)]
    outs = _row_spec(BLKE, 128)
    shp = jax.ShapeDtypeStruct((EPAD, 128), _F32)
    return _tc_call(_enc_edge_body, g, specs, outs, shp)(
        gs, gr, amat, wn, mn, ce, w2, b2, w3, b3)


def _edge_blk_call(el, gs, gr, w1, b1, w2, b2, w3, b3):
    g = EPAD // BLKE
    specs = ([_row_spec(BLKE, 128)] * 3 +
             [_w_spec(128, 128), _w_spec(1, 128), _w_spec(128, 128),
              _w_spec(1, 128), _w_spec(128, 128), _w_spec(1, 128)])
    outs = [_row_spec(BLKE, 128)] * 2
    shp = [jax.ShapeDtypeStruct((EPAD, 128), _F32)] * 2
    return _tc_call(_edge_blk_body, g, specs, outs, shp)(
        el, gs, gr, w1, b1, w2, b2, w3, b3)


def _node_blk_call(nl, p0, p1, w1n, w1a, b1, w2, b2, w3, b3, w1s=None,
                   w1r=None):
    g = NPAD // BLKN
    wspecs = [_w_spec(128, 128), _w_spec(128, 128), _w_spec(1, 128),
              _w_spec(128, 128), _w_spec(1, 128), _w_spec(128, 128),
              _w_spec(1, 128)]
    if w1s is not None:
        specs = [_row_spec(BLKN, 128)] * 3 + wspecs + [_w_spec(128, 128)] * 2
        outs = [_row_spec(BLKN, 128)] * 3
        shp = [jax.ShapeDtypeStruct((NPAD, 128), _F32)] * 3
        return _tc_call(_node_blk_body_proj, g, specs, outs, shp)(
            nl, p0, p1, w1n, w1a, b1, w2, b2, w3, b3, w1s, w1r)
    specs = [_row_spec(BLKN, 128)] * 3 + wspecs
    outs = _row_spec(BLKN, 128)
    shp = jax.ShapeDtypeStruct((NPAD, 128), _F32)
    return _tc_call(_node_blk_body_last, g, specs, outs, shp)(
        nl, p0, p1, w1n, w1a, b1, w2, b2, w3, b3)


def _dec_call(nl, c1m, a1m, maskf, w1, b1, w2, b2, w3p, b3p):
    g = NPAD // BLKN
    specs = [_row_spec(BLKN, 128), _row_spec(BLKN, 8), _row_spec(BLKN, 8),
             _row_spec(BLKN, 1),
             _w_spec(128, 128), _w_spec(1, 128), _w_spec(128, 128),
             _w_spec(1, 128), _w_spec(128, 8), _w_spec(1, 8)]
    outs = _row_spec(BLKN, 8)
    shp = jax.ShapeDtypeStruct((NPAD, 8), _F32)
    return _tc_call(_dec_body, g, specs, outs, shp)(
        nl, c1m, a1m, maskf, w1, b1, w2, b2, w3p, b3p)


def _sc_mesh():
    return plsc.VectorSubcoreMesh(core_axis_name="c", subcore_axis_name="s")


@functools.lru_cache(maxsize=None)
def _make_sc_gather(d):
    @functools.partial(
        pl.kernel,
        out_type=(jax.ShapeDtypeStruct((EPAD, d), _F32),
                  jax.ShapeDtypeStruct((EPAD, d), _F32)),
        mesh=_sc_mesh(),
        scratch_types=[
            pltpu.VMEM((CPW, CH), jnp.int32),
            pltpu.VMEM((CPW, CH), jnp.int32),
            pltpu.VMEM((CH, d), _F32),
            pltpu.VMEM((CH, d), _F32),
            pltpu.VMEM((CH, d), _F32),
            pltpu.VMEM((CH, d), _F32),
            pltpu.SemaphoreType.DMA,
            pltpu.SemaphoreType.DMA,
            pltpu.SemaphoreType.DMA,
            pltpu.SemaphoreType.DMA,
        ],
    )
    def gather(tab_s, tab_r, sidx, ridx, gs_o, gr_o,
               sidx_v, ridx_v, bs0, bs1, br0, br1, e0, e1, e2, e3):
        cid = lax.axis_index("c")
        sid = lax.axis_index("s")
        wid = sid * 2 + cid
        row0 = wid * CPW
        base = wid * EPW
        pltpu.sync_copy(sidx.at[pl.ds(row0, CPW)], sidx_v)
        pltpu.sync_copy(ridx.at[pl.ds(row0, CPW)], ridx_v)

        def step(i, carry):
            c0 = 2 * i
            c1 = c0 + 1
            g0 = pltpu.async_copy(tab_s.at[sidx_v.at[c0]], bs0, e0)
            g1 = pltpu.async_copy(tab_r.at[ridx_v.at[c0]], br0, e1)
            g2 = pltpu.async_copy(tab_s.at[sidx_v.at[c1]], bs1, e2)
            g3 = pltpu.async_copy(tab_r.at[ridx_v.at[c1]], br1, e3)
            g0.wait()
            pltpu.sync_copy(bs0, gs_o.at[pl.ds(base + c0 * CH, CH)])
            g1.wait()
            pltpu.sync_copy(br0, gr_o.at[pl.ds(base + c0 * CH, CH)])
            g2.wait()
            pltpu.sync_copy(bs1, gs_o.at[pl.ds(base + c1 * CH, CH)])
            g3.wait()
            pltpu.sync_copy(br1, gr_o.at[pl.ds(base + c1 * CH, CH)])
            return carry

        lax.fori_loop(0, CPW // 2, step, 0)

    return gather


def _make_sc_scatter():
    @functools.partial(
        pl.kernel,
        out_type=jax.ShapeDtypeStruct((2, NPAD, LATENT), _F32),
        mesh=_sc_mesh(),
        scratch_types=[
            pltpu.VMEM((CPW, CH), jnp.int32),
            pltpu.VMEM((CH, LATENT), _F32),
            pltpu.VMEM((CH, LATENT), _F32),
            pltpu.VMEM_SHARED((NPAD, LATENT), _F32),
            pltpu.SemaphoreType.DMA,
            pltpu.SemaphoreType.DMA,
        ],
    )
    def scatter(ne, rscat, ztile, out, idx_v, b0, b1, acc, e0, e1):
        cid = lax.axis_index("c")
        sid = lax.axis_index("s")
        wid = sid * 2 + cid
        row0 = wid * CPW
        base = wid * EPW
        pltpu.sync_copy(rscat.at[pl.ds(row0, CPW)], idx_v)
        pltpu.sync_copy(ztile, acc.at[pl.ds(sid * ROWS_PER_TILE,
                                            ROWS_PER_TILE)])
        plsc.subcore_barrier()

        def step(i, carry):
            c0 = 2 * i
            c1 = c0 + 1
            l0 = pltpu.async_copy(ne.at[pl.ds(base + c0 * CH, CH)], b0, e0)
            l1 = pltpu.async_copy(ne.at[pl.ds(base + c1 * CH, CH)], b1, e1)
            l0.wait()
            pltpu.sync_copy(b0, acc.at[idx_v.at[c0]], add=True)
            l1.wait()
            pltpu.sync_copy(b1, acc.at[idx_v.at[c1]], add=True)
            return carry

        lax.fori_loop(0, CPW // 2, step, 0)
        plsc.subcore_barrier()
        pltpu.sync_copy(acc.at[pl.ds(sid * ROWS_PER_TILE, ROWS_PER_TILE)],
                        out.at[cid].at[pl.ds(sid * ROWS_PER_TILE,
                                             ROWS_PER_TILE)])

    return scatter


def _sc_gather(tab_s, tab_r, sidx2d, ridx2d, d):
    return _make_sc_gather(d)(tab_s, tab_r, sidx2d, ridx2d)


_SC_SCATTER = None


def _sc_scatter(ne, rscat2d, ztile):
    global _SC_SCATTER
    if _SC_SCATTER is None:
        _SC_SCATTER = _make_sc_scatter()
    return _SC_SCATTER(ne, rscat2d, ztile)


def _pad_rows(x, n):
    return jnp.pad(x, ((0, n - x.shape[0]),) + ((0, 0),) * (x.ndim - 1))


def kernel(world_pos, prev_world_pos, mesh_pos, node_type, edge_index,
           params):
    f32 = _F32
    senders = edge_index[0].astype(jnp.int32)
    receivers = edge_index[1].astype(jnp.int32)

    sidx2d = jnp.pad(senders, (0, EPAD - N_EDGES)).reshape(EPAD // CH, CH)
    ridx2d = jnp.pad(receivers, (0, EPAD - N_EDGES)).reshape(EPAD // CH, CH)
    rscat2d = jnp.pad(receivers, (0, EPAD - N_EDGES),
                      constant_values=TRASH).reshape(EPAD // CH, CH)

    wp = world_pos.astype(f32)
    pp = prev_world_pos.astype(f32)
    mp = mesh_pos.astype(f32)

    tpos = _pad_rows(jnp.concatenate(
        [wp, mp, jnp.zeros((N_NODES, 122), f32)], axis=1), NPAD)
    wpp = _pad_rows(jnp.concatenate(
        [wp, pp, jnp.zeros((N_NODES, 2), f32)], axis=1), NPAD)
    tpe = _pad_rows(node_type.astype(jnp.int32)[:, None], NPAD)

    p = params
    nm, ns = p['node_norm']
    (w1, b1), (w2, b2), (w3, b3) = p['enc_node']
    w1p = w1 / ns[:, None]
    c1 = (b1 - (nm / ns) @ w1)[None, :]
    bwp = jnp.zeros((8, 128), f32).at[0:3].set(w1p[0:3]).at[3:6].set(-w1p[0:3])
    w1t = jnp.zeros((16, 128), f32).at[0:N_TYPES].set(w1p[3:3 + N_TYPES])
    enc_node_w = (bwp, w1t, c1, w2, b2[None, :], w3, b3[None, :])

    mm, ms = p['mesh_norm']
    (ew1, eb1), (ew2, eb2), (ew3, eb3) = p['enc_edge']
    ew1p = ew1 / ms[:, None]
    ce = (eb1 - (mm / ms) @ ew1)[None, :]
    amat = jnp.zeros((128, 128), f32).at[0:3].set(ew1p[0:3]).at[3:6].set(
        ew1p[4:7])
    wn = ew1p[3][None, :]
    mnr = ew1p[7][None, :]
    enc_edge_w = (amat, wn, mnr, ce, ew2, eb2[None, :], ew3, eb3[None, :])

    blocks = []
    for blk in p['blocks']:
        (bw1, bb1), (bw2, bb2), (bw3, bb3) = blk['edge']
        (nw1, nb1), (nw2, nb2), (nw3, nb3) = blk['node']
        blocks.append(dict(
            w1e=bw1[0:128], w1s=bw1[128:256], w1r=bw1[256:384],
            eb1=bb1[None, :], ew2=bw2, eb2=bb2[None, :], ew3=bw3,
            eb3=bb3[None, :],
            w1n=nw1[0:128], w1a=nw1[128:256], nb1=nb1[None, :],
            nw2=nw2, nb2=nb2[None, :], nw3=nw3, nb3=nb3[None, :]))

    (dw1, db1), (dw2, db2), (dw3, db3) = p['dec']
    om, osc = p['out_norm']
    w3p = jnp.zeros((128, 8), f32).at[:, 0:3].set(dw3 * osc[None, :])
    b3p = jnp.zeros((1, 8), f32).at[0, 0:3].set(db3 * osc + om)
    maskf = _pad_rows((node_type == 0).astype(f32)[:, None], NPAD)
    c1m = _pad_rows(jnp.concatenate(
        [2.0 * wp - pp, wp, jnp.zeros((N_NODES, 2), f32)], axis=1), NPAD)
    a1m = _pad_rows(jnp.concatenate(
        [wp, pp, jnp.zeros((N_NODES, 2), f32)], axis=1), NPAD)

    ztile = jnp.zeros((ROWS_PER_TILE, LATENT), f32)

    b0 = blocks[0]
    nl, ps, pr = _enc_node_call(wpp, tpe, *enc_node_w, b0['w1s'], b0['w1r'])
    gs16, gr16 = _sc_gather(tpos, tpos, sidx2d, ridx2d, 128)
    el = _enc_edge_call(gs16, gr16, *enc_edge_w)

    for b in range(N_BLOCKS):
        bw = blocks[b]
        gs, gr = _sc_gather(ps, pr, sidx2d, ridx2d, 128)
        ne, el = _edge_blk_call(el, gs, gr, bw['w1e'], bw['eb1'], bw['ew2'],
                                bw['eb2'], bw['ew3'], bw['eb3'])
        part = _sc_scatter(ne, rscat2d, ztile)
        p0, p1 = part[0], part[1]
        if b < N_BLOCKS - 1:
            nb = blocks[b + 1]
            nl, ps, pr = _node_blk_call(
                nl, p0, p1, bw['w1n'], bw['w1a'], bw['nb1'], bw['nw2'],
                bw['nb2'], bw['nw3'], bw['nb3'], nb['w1s'], nb['w1r'])
        else:
            nl = _node_blk_call(
                nl, p0, p1, bw['w1n'], bw['w1a'], bw['nb1'], bw['nw2'],
                bw['nb2'], bw['nw3'], bw['nb3'])

    out8 = _dec_call(nl, c1m, a1m, maskf, dw1, db1[None, :], dw2,
                     db2[None, :], w3p, b3p)
    return out8[:N_NODES, 0:6]
